# baseline (speedup 1.0000x reference)
import functools

import jax
import jax.numpy as jnp
from jax import lax
from jax.experimental import pallas as pl
from jax.experimental.pallas import tpu as pltpu
from jax.experimental.pallas import tpu_sc as plsc

NUM_EXPERTS = 16
TOP_K = 2
TOKENS = 16384
D_MODEL = 2048

BLOCK_T = 1024
D_HALF = D_MODEL // 2

NUM_WORKERS = 32
TOK_W = TOKENS // NUM_WORKERS


def _logits_kernel(xa_ref, xb_ref, wta_ref, wtb_ref, lt_ref):
    logits = jnp.dot(
        xa_ref[...], wta_ref[...], preferred_element_type=jnp.float32
    ) + jnp.dot(xb_ref[...], wtb_ref[...], preferred_element_type=jnp.float32)
    lt_ref[...] = logits.T


def _tc_logits(x, wt):
    grid = (TOKENS // BLOCK_T,)
    return pl.pallas_call(
        _logits_kernel,
        grid=grid,
        in_specs=[
            pl.BlockSpec((BLOCK_T, D_HALF), lambda i: (i, 0)),
            pl.BlockSpec((BLOCK_T, D_HALF), lambda i: (i, 1)),
            pl.BlockSpec((D_HALF, NUM_EXPERTS), lambda i: (0, 0)),
            pl.BlockSpec((D_HALF, NUM_EXPERTS), lambda i: (0, 0)),
        ],
        out_specs=[
            pl.BlockSpec((NUM_EXPERTS, BLOCK_T), lambda i: (0, i)),
        ],
        out_shape=[
            jax.ShapeDtypeStruct((NUM_EXPERTS, TOKENS), jnp.float32),
        ],
        compiler_params=pltpu.CompilerParams(
            dimension_semantics=("parallel",),
        ),
    )(x, x, wt[:D_HALF], wt[D_HALF:])[0]


def _sc_route(lt_hbm, ew_out, meta_out, ltv, ewv, metav):
    c = lax.axis_index("c")
    s = lax.axis_index("s")
    wid = s * 2 + c
    tok_base = wid * TOK_W
    pltpu.sync_copy(lt_hbm.at[:, pl.ds(tok_base, TOK_W)], ltv)

    iota = lax.iota(jnp.int32, 16)
    iotaf = iota.astype(jnp.float32)
    sixteenf = jnp.full((16,), 16.0, jnp.float32)
    neg_inf = jnp.float32(-jnp.inf)

    def token_body(t, carry):
        tvec = jnp.full((16,), t, jnp.int32)
        lg = plsc.load_gather(ltv, [iota, tvec])

        def bcast_reduce(v, op):
            for st in (1, 2, 4, 8):
                metav[t] = v
                perm = plsc.load_gather(
                    metav, [tvec, jnp.bitwise_xor(iota, st)]
                )
                v = op(v, perm)
            return v

        m0 = bcast_reduce(lg, jnp.maximum)
        i0 = bcast_reduce(jnp.where(lg == m0, iotaf, sixteenf), jnp.minimum)
        lm = jnp.where(iotaf == i0, neg_inf, lg)
        m1 = bcast_reduce(lm, jnp.maximum)
        i1 = bcast_reduce(jnp.where(lm == m1, iotaf, sixteenf), jnp.minimum)

        ev = jnp.exp(m1 - m0)
        wn0 = 1.0 / (1.0 + ev)
        wn1 = ev / (1.0 + ev)
        ewv[t] = (
            wn0 * (iotaf == i0).astype(jnp.float32)
            + wn1 * (iotaf == i1).astype(jnp.float32)
        )
        metav[t] = (
            wn0 * (iota == 0).astype(jnp.float32)
            + wn1 * (iota == 1).astype(jnp.float32)
            + i0 * (iota == 2).astype(jnp.float32)
            + i1 * (iota == 3).astype(jnp.float32)
        )
        return carry

    lax.fori_loop(0, TOK_W, token_body, jnp.int32(0))

    pltpu.sync_copy(ewv, ew_out.at[pl.ds(tok_base, TOK_W)])
    pltpu.sync_copy(metav, meta_out.at[pl.ds(tok_base, TOK_W)])


def _sc_part(lt):
    mesh = plsc.VectorSubcoreMesh(core_axis_name="c", subcore_axis_name="s")
    sc = functools.partial(
        pl.kernel,
        mesh=mesh,
        out_type=[
            jax.ShapeDtypeStruct((TOKENS, 16), jnp.float32),
            jax.ShapeDtypeStruct((TOKENS, 16), jnp.float32),
        ],
        scratch_types=[
            pltpu.VMEM((NUM_EXPERTS, TOK_W), jnp.float32),
            pltpu.VMEM((TOK_W, 16), jnp.float32),
            pltpu.VMEM((TOK_W, 16), jnp.float32),
        ],
        compiler_params=pltpu.CompilerParams(use_tc_tiling_on_sc=False, needs_layout_passes=False),
    )(_sc_route)
    return sc(lt)


@jax.jit
def kernel(inputs, W):
    x = inputs.astype(jnp.float32)
    wt = W.T
    lt = _tc_logits(x, wt)
    ew_sc, meta_sc = _sc_part(lt)

    w = meta_sc[:, 0:2]
    i = meta_sc[:, 2:4].astype(jnp.int32)
    ew = ew_sc.T
    return w, i, ew


# speedup vs baseline: 2.1055x; 2.1055x over previous
import functools

import jax
import jax.numpy as jnp
from jax import lax
from jax.experimental import pallas as pl
from jax.experimental.pallas import tpu as pltpu
from jax.experimental.pallas import tpu_sc as plsc

NUM_EXPERTS = 16
TOP_K = 2
TOKENS = 16384
D_MODEL = 2048

BLOCK_T = 1024
D_HALF = D_MODEL // 2

NUM_WORKERS = 32
TOK_W = TOKENS // NUM_WORKERS


def _logits_kernel(xa_ref, xb_ref, wta_ref, wtb_ref, lt_ref):
    logits = jnp.dot(
        xa_ref[...], wta_ref[...], preferred_element_type=jnp.float32
    ) + jnp.dot(xb_ref[...], wtb_ref[...], preferred_element_type=jnp.float32)
    lt_ref[...] = logits.T


def _tc_logits(x, wt):
    grid = (TOKENS // BLOCK_T,)
    return pl.pallas_call(
        _logits_kernel,
        grid=grid,
        in_specs=[
            pl.BlockSpec((BLOCK_T, D_HALF), lambda i: (i, 0)),
            pl.BlockSpec((BLOCK_T, D_HALF), lambda i: (i, 1)),
            pl.BlockSpec((D_HALF, NUM_EXPERTS), lambda i: (0, 0)),
            pl.BlockSpec((D_HALF, NUM_EXPERTS), lambda i: (0, 0)),
        ],
        out_specs=[
            pl.BlockSpec((NUM_EXPERTS, BLOCK_T), lambda i: (0, i)),
        ],
        out_shape=[
            jax.ShapeDtypeStruct((NUM_EXPERTS, TOKENS), jnp.float32),
        ],
        compiler_params=pltpu.CompilerParams(
            dimension_semantics=("parallel",),
        ),
    )(x, x, wt[:D_HALF], wt[D_HALF:])[0]


def _sc_route(lt_hbm, ew_out, meta_out, ltv, ewv, mv):
    c = lax.axis_index("c")
    s = lax.axis_index("s")
    wid = s * 2 + c
    tok_base = wid * TOK_W
    pltpu.sync_copy(lt_hbm.at[:, pl.ds(tok_base, TOK_W)], ltv)

    neg_inf = jnp.float32(-jnp.inf)

    def group_body(tg, carry):
        col = tg * 16
        lg = [ltv[e, pl.ds(col, 16)] for e in range(NUM_EXPERTS)]

        m0 = lg[0]
        for e in range(1, NUM_EXPERTS):
            m0 = jnp.maximum(m0, lg[e])
        i0 = jnp.full((16,), 0.0, jnp.float32)
        for e in range(NUM_EXPERTS - 1, -1, -1):
            i0 = jnp.where(lg[e] == m0, jnp.float32(e), i0)

        lm = [jnp.where(i0 == jnp.float32(e), neg_inf, lg[e])
              for e in range(NUM_EXPERTS)]
        m1 = lm[0]
        for e in range(1, NUM_EXPERTS):
            m1 = jnp.maximum(m1, lm[e])
        i1 = jnp.full((16,), 0.0, jnp.float32)
        for e in range(NUM_EXPERTS - 1, -1, -1):
            i1 = jnp.where(lm[e] == m1, jnp.float32(e), i1)

        ev = jnp.exp(m1 - m0)
        wn0 = 1.0 / (1.0 + ev)
        wn1 = ev / (1.0 + ev)
        for e in range(NUM_EXPERTS):
            ef = jnp.float32(e)
            ewv[e, pl.ds(col, 16)] = (
                wn0 * (i0 == ef).astype(jnp.float32)
                + wn1 * (i1 == ef).astype(jnp.float32)
            )
        mv[0, pl.ds(col, 16)] = wn0
        mv[1, pl.ds(col, 16)] = wn1
        mv[2, pl.ds(col, 16)] = i0
        mv[3, pl.ds(col, 16)] = i1
        return carry

    lax.fori_loop(0, TOK_W // 16, group_body, jnp.int32(0))

    pltpu.sync_copy(ewv, ew_out.at[:, pl.ds(tok_base, TOK_W)])
    pltpu.sync_copy(mv, meta_out.at[:, pl.ds(tok_base, TOK_W)])


def _sc_part(lt):
    mesh = plsc.VectorSubcoreMesh(core_axis_name="c", subcore_axis_name="s")
    sc = functools.partial(
        pl.kernel,
        mesh=mesh,
        out_type=[
            jax.ShapeDtypeStruct((NUM_EXPERTS, TOKENS), jnp.float32),
            jax.ShapeDtypeStruct((4, TOKENS), jnp.float32),
        ],
        scratch_types=[
            pltpu.VMEM((NUM_EXPERTS, TOK_W), jnp.float32),
            pltpu.VMEM((NUM_EXPERTS, TOK_W), jnp.float32),
            pltpu.VMEM((4, TOK_W), jnp.float32),
        ],
        compiler_params=pltpu.CompilerParams(
            use_tc_tiling_on_sc=False, needs_layout_passes=False
        ),
    )(_sc_route)
    return sc(lt)


@jax.jit
def kernel(inputs, W):
    x = inputs.astype(jnp.float32)
    wt = W.T
    lt = _tc_logits(x, wt)
    ew_sc, meta_sc = _sc_part(lt)

    w = meta_sc[0:2].T
    i = meta_sc[2:4].T.astype(jnp.int32)
    return w, i, ew_sc


# speedup vs baseline: 3.3575x; 1.5947x over previous
import jax
import jax.numpy as jnp
from jax.experimental import pallas as pl
from jax.experimental.pallas import tpu as pltpu

NUM_EXPERTS = 16
TOP_K = 2
TOKENS = 16384
D_MODEL = 2048

BLOCK_T = 1024
D_HALF = D_MODEL // 2


def _router_kernel(xa_ref, xb_ref, wta_ref, wtb_ref, w_out_ref, i_out_ref, ew_ref):
    logits = jnp.dot(
        xa_ref[...], wta_ref[...], preferred_element_type=jnp.float32
    ) + jnp.dot(xb_ref[...], wtb_ref[...], preferred_element_type=jnp.float32)
    lt = logits.T

    sub = jax.lax.broadcasted_iota(jnp.int32, lt.shape, 0)

    l0 = jnp.max(lt, axis=0, keepdims=True)
    i0 = jnp.argmax(lt, axis=0, keepdims=True)
    masked = jnp.where(sub == i0, -jnp.inf, lt)
    l1 = jnp.max(masked, axis=0, keepdims=True)
    i1 = jnp.argmax(masked, axis=0, keepdims=True)

    e1 = jnp.exp(l1 - l0)
    wn0 = 1.0 / (1.0 + e1)
    wn1 = e1 / (1.0 + e1)

    w_out_ref[...] = jnp.concatenate([wn0, wn1], axis=0)
    i_out_ref[...] = jnp.concatenate([i0, i1], axis=0).astype(jnp.int32)

    ew_ref[...] = wn0 * (sub == i0).astype(jnp.float32) + wn1 * (
        sub == i1
    ).astype(jnp.float32)


@jax.jit
def kernel(inputs, W):
    T, D = inputs.shape
    E = W.shape[0]
    x = inputs.astype(jnp.float32)
    wt = W.T
    grid = (T // BLOCK_T,)
    w_out, i_out, ew = pl.pallas_call(
        _router_kernel,
        grid=grid,
        in_specs=[
            pl.BlockSpec((BLOCK_T, D_HALF), lambda i: (i, 0)),
            pl.BlockSpec((BLOCK_T, D_HALF), lambda i: (i, 1)),
            pl.BlockSpec((D_HALF, E), lambda i: (0, 0)),
            pl.BlockSpec((D_HALF, E), lambda i: (0, 0)),
        ],
        out_specs=[
            pl.BlockSpec((TOP_K, BLOCK_T), lambda i: (0, i)),
            pl.BlockSpec((TOP_K, BLOCK_T), lambda i: (0, i)),
            pl.BlockSpec((E, BLOCK_T), lambda i: (0, i)),
        ],
        out_shape=[
            jax.ShapeDtypeStruct((TOP_K, T), jnp.float32),
            jax.ShapeDtypeStruct((TOP_K, T), jnp.int32),
            jax.ShapeDtypeStruct((E, T), jnp.float32),
        ],
        compiler_params=pltpu.CompilerParams(
            dimension_semantics=("parallel",),
        ),
    )(x, x, wt[:D_HALF], wt[D_HALF:])
    return w_out.T, i_out.T, ew
